# Initial kernel scaffold; baseline (speedup 1.0000x reference)
import jax
import jax.numpy as jnp
from jax.experimental import pallas as pl


def kernel(wf, labels):
    raise NotImplementedError("write your pallas kernel here")


# speedup vs baseline: 1.6855x; 1.6855x over previous
import jax
import jax.numpy as jnp
from jax.experimental import pallas as pl
from jax.experimental.pallas import tpu as pltpu

_S = 30.0
_M = 0.4


def _loss_body(lab_ref, wf_ref, out_ref):
    x = wf_ref[...]
    labs = lab_ref[0, 0, :]
    br, c = x.shape
    cols = jax.lax.broadcasted_iota(jnp.int32, (br, c), 1)
    mask = cols == labs[:, None]
    xm = jnp.where(mask, x - _M, x)
    rowmax = jnp.max(xm, axis=1, keepdims=True)
    e = jnp.exp((xm - rowmax) * _S)
    s = jnp.sum(e, axis=1, keepdims=True)
    t = jnp.sum(jnp.where(mask, x, 0.0), axis=1, keepdims=True)
    loss = _S * (t - _M) - (_S * rowmax + jnp.log(s))
    out_ref[0, 0, :] = loss[:, 0]


def kernel(wf, labels):
    B, C = wf.shape
    BR = 128
    G = B // BR
    labs = labels.astype(jnp.int32).reshape(G, 1, BR)
    out = pl.pallas_call(
        _loss_body,
        grid=(G,),
        in_specs=[
            pl.BlockSpec((1, 1, BR), lambda i: (i, 0, 0)),
            pl.BlockSpec((BR, C), lambda i: (i, 0)),
        ],
        out_specs=pl.BlockSpec((1, 1, BR), lambda i: (i, 0, 0)),
        out_shape=jax.ShapeDtypeStruct((G, 1, BR), jnp.float32),
        compiler_params=pltpu.CompilerParams(
            dimension_semantics=("parallel",),
            vmem_limit_bytes=50 * 1024 * 1024,
        ),
        name="cosface_loss",
    )(labs, wf)
    return -jnp.mean(out.reshape(B))


# speedup vs baseline: 2.5262x; 1.4988x over previous
import jax
import jax.numpy as jnp
from jax.experimental import pallas as pl
from jax.experimental.pallas import tpu as pltpu

_S = 30.0
_M = 0.4
_LOG2E = 1.4426950408889634
_C1 = _S * _LOG2E
_BR = 128


def _loss_body(lab_ref, wf_ref, out_ref, trow_ref):
    i = pl.program_id(0)
    x = wf_ref[...]
    m0 = jnp.max(x, axis=1, keepdims=True)

    lane_iota = jax.lax.broadcasted_iota(jnp.int32, (1, 128), 1)
    for r in range(_BR):
        col = lab_ref[i * _BR + r]
        base = pl.multiple_of((col >> 7) << 7, 128)
        v = wf_ref[pl.ds(r, 1), pl.ds(base, 128)]
        trow_ref[pl.ds(r, 1), :] = jnp.where(lane_iota == (col & 127), v, 0.0)
    t = jnp.sum(trow_ref[...], axis=1, keepdims=True)

    pm0 = m0 * _C1
    e = jnp.exp2(x * _C1 - pm0)
    s_all = jnp.sum(e, axis=1, keepdims=True)

    e1 = jnp.exp2(t * _C1 - pm0)
    s_adj = s_all - e1 + e1 * (2.0 ** (-_C1 * _M))
    loss = _S * (t - _M) - (_S * m0 + jnp.log(s_adj))
    out_ref[0, 0, :] = loss[:, 0]


def kernel(wf, labels):
    B, C = wf.shape
    G = B // _BR
    labs = labels.astype(jnp.int32)
    out = pl.pallas_call(
        _loss_body,
        grid_spec=pltpu.PrefetchScalarGridSpec(
            num_scalar_prefetch=1,
            grid=(G,),
            in_specs=[pl.BlockSpec((_BR, C), lambda i, lab: (i, 0))],
            out_specs=pl.BlockSpec((1, 1, _BR), lambda i, lab: (i, 0, 0)),
            scratch_shapes=[pltpu.VMEM((_BR, 128), jnp.float32)],
        ),
        out_shape=jax.ShapeDtypeStruct((G, 1, _BR), jnp.float32),
        compiler_params=pltpu.CompilerParams(
            dimension_semantics=("parallel",),
            vmem_limit_bytes=50 * 1024 * 1024,
        ),
        name="cosface_loss",
    )(labs, wf)
    return -jnp.mean(out.reshape(B))
